# Initial kernel scaffold; baseline (speedup 1.0000x reference)
import jax
import jax.numpy as jnp
from jax.experimental import pallas as pl


def kernel(input_ids, embed, wq, wk, wv, wo, w1, w2, w3, n1, n2, nf):
    raise NotImplementedError("write your pallas kernel here")


# speedup vs baseline: 1.3850x; 1.3850x over previous
import functools
import math

import numpy as np
import jax
import jax.numpy as jnp
from jax.experimental import pallas as pl
from jax.experimental.pallas import tpu as pltpu
from jax.experimental.pallas import tpu_sc as plsc

D = 1024
H = 16
DH = 64
FF = 4096
T = 2048
NUM_BLOCKS = 32
TOPK_BLOCKS = 8
BS = T // NUM_BLOCKS
LOW_PASS = 0.25


def _build_consts():
    fn = T // 2 + 1
    cutoff = max(1, int(LOW_PASS * fn))
    t = np.arange(T, dtype=np.float64)
    cols = [np.full(T, 1.0 / np.sqrt(T))]
    for f in range(1, cutoff):
        w = 2.0 * np.pi * f * t / T
        cols.append(np.sqrt(2.0 / T) * np.cos(w))
        cols.append(np.sqrt(2.0 / T) * np.sin(w))
    A = np.stack(cols, axis=1)
    pad = (-A.shape[1]) % 128
    A = np.concatenate([A, np.zeros((T, pad))], axis=1)
    SB = np.zeros((T, NUM_BLOCKS))
    SB[np.arange(T), np.arange(T) // BS] = 1.0 / BS
    inv_freq = 1.0 / (10000.0 ** (np.arange(0, DH, 2, dtype=np.float64) / DH))
    freqs = t[:, None] * inv_freq[None, :]
    emb = np.concatenate([freqs, freqs], axis=1)
    cos2 = np.tile(np.cos(emb), (1, 2))
    sin2 = np.tile(np.sin(emb), (1, 2))
    f32 = np.float32
    return A.astype(f32), A.T.copy().astype(f32), SB.astype(f32), cos2.astype(f32), sin2.astype(f32)


_A_NP, _AT_NP, _SB_NP, _COS2_NP, _SIN2_NP = _build_consts()


def _gather_sc(embed, ids):
    win = 128
    split = 4
    dsub = D // split
    ids4 = (ids.reshape(T, 1) * split
            + jnp.arange(split, dtype=jnp.int32).reshape(1, split)
            ).reshape(1, T * split)
    table = embed.reshape(embed.shape[0] * split, dsub)
    mesh = plsc.VectorSubcoreMesh(core_axis_name="c", subcore_axis_name="s")

    @functools.partial(
        pl.kernel,
        out_type=jax.ShapeDtypeStruct((T * split, dsub), embed.dtype),
        mesh=mesh,
    )
    def body(x_hbm, i_hbm, o_hbm):
        def inner(i_vmem, o_vmem):
            pltpu.sync_copy(x_hbm.at[i_vmem.at[0]], o_vmem)

        pltpu.emit_pipeline(
            inner,
            grid=(T * split // win,),
            in_specs=[pl.BlockSpec((1, win), lambda i: (0, i))],
            out_specs=[pl.BlockSpec((win, dsub), lambda i: (i, 0))],
            core_axis_name=("c", "s"),
            dimension_semantics=(pltpu.PARALLEL,),
        )(i_hbm, o_hbm)

    return body(table, ids4).reshape(T, D)


def _select_kernel(x_ref, a_ref, at_ref, sb_ref, sel_ref):
    x = x_ref[...]
    g = jnp.dot(at_ref[...], x, preferred_element_type=jnp.float32)
    lp = jnp.dot(a_ref[...], g, preferred_element_type=jnp.float32)
    nlp = jnp.sqrt(jnp.sum(lp * lp, axis=1, keepdims=True))
    nx = jnp.sqrt(jnp.sum(x * x, axis=1, keepdims=True))
    rho = nlp / (nx + 1e-6)
    bq = jax.lax.dot_general(
        rho, sb_ref[...], (((0,), (0,)), ((), ())),
        preferred_element_type=jnp.float32)
    iota = jax.lax.broadcasted_iota(
        jnp.int32, (1, NUM_BLOCKS), 1).astype(jnp.float32)
    iota8 = jax.lax.broadcasted_iota(
        jnp.int32, (1, TOPK_BLOCKS), 1).astype(jnp.float32)
    sel = jnp.zeros((1, TOPK_BLOCKS), jnp.float32)
    b = bq
    for j in range(TOPK_BLOCKS):
        mx = jnp.max(b, axis=1, keepdims=True)
        idx = jnp.min(jnp.where(b == mx, iota, jnp.float32(1e9)), axis=1,
                      keepdims=True)
        b = jnp.where(iota == idx, jnp.float32(-1.0), b)
        sel = sel + idx * (iota8 == jnp.float32(j)).astype(jnp.float32)
    sel_ref[...] = sel


def _select(x):
    consts = (jnp.asarray(_A_NP), jnp.asarray(_AT_NP), jnp.asarray(_SB_NP))
    return pl.pallas_call(
        _select_kernel,
        out_shape=jax.ShapeDtypeStruct((1, TOPK_BLOCKS), jnp.float32),
        in_specs=[
            pl.BlockSpec((T, D), lambda: (0, 0)),
            pl.BlockSpec((T, 512), lambda: (0, 0)),
            pl.BlockSpec((512, T), lambda: (0, 0)),
            pl.BlockSpec((T, NUM_BLOCKS), lambda: (0, 0)),
        ],
        out_specs=pl.BlockSpec((1, TOPK_BLOCKS), lambda: (0, 0)),
    )(x, *consts)


def _rmsnorm_kernel(x_ref, w_ref, o_ref):
    x = x_ref[...]
    inv = jax.lax.rsqrt(jnp.mean(x * x, axis=1, keepdims=True) + 1e-6)
    o_ref[...] = x * inv * w_ref[...]


def _rmsnorm(x, w):
    return pl.pallas_call(
        _rmsnorm_kernel,
        grid=(8,),
        out_shape=jax.ShapeDtypeStruct((T, D), jnp.float32),
        in_specs=[
            pl.BlockSpec((T // 8, D), lambda i: (i, 0)),
            pl.BlockSpec((1, D), lambda i: (0, 0)),
        ],
        out_specs=pl.BlockSpec((T // 8, D), lambda i: (i, 0)),
    )(x, w.reshape(1, D))


def _rot_half2(s):
    return jnp.concatenate(
        [-s[:, 32:64], s[:, 0:32], -s[:, 96:128], s[:, 64:96]], axis=1)


def _qkv_kernel(h_ref, wq_ref, wk_ref, wv_ref, cos_ref, sin_ref,
                q_ref, k_ref, v_ref):
    h = h_ref[...]
    cos = cos_ref[...]
    sin = sin_ref[...]
    q = jnp.dot(h, wq_ref[...], preferred_element_type=jnp.float32)
    k = jnp.dot(h, wk_ref[...], preferred_element_type=jnp.float32)
    v = jnp.dot(h, wv_ref[...], preferred_element_type=jnp.float32)
    q_ref[...] = q * cos + _rot_half2(q) * sin
    k_ref[...] = k * cos + _rot_half2(k) * sin
    v_ref[...] = v


def _qkv(h, wq, wk, wv):
    cos2, sin2 = jnp.asarray(_COS2_NP), jnp.asarray(_SIN2_NP)
    sds = jax.ShapeDtypeStruct((T, D), jnp.float32)
    wspec = pl.BlockSpec((D, 128), lambda j: (0, j))
    ospec = pl.BlockSpec((T, 128), lambda j: (0, j))
    return pl.pallas_call(
        _qkv_kernel,
        grid=(D // 128,),
        out_shape=(sds, sds, sds),
        in_specs=[
            pl.BlockSpec((T, D), lambda j: (0, 0)),
            wspec, wspec, wspec,
            pl.BlockSpec((T, 128), lambda j: (0, 0)),
            pl.BlockSpec((T, 128), lambda j: (0, 0)),
        ],
        out_specs=(ospec, ospec, ospec),
    )(h, wq, wk, wv, cos2, sin2)


_NSEL = TOPK_BLOCKS * BS
_QT = 256


def _attn_kernel(sel_ref, q_ref, k_ref, v_ref, o_ref, kc, vc, kpos):
    for jj in range(TOPK_BLOCKS):
        s = sel_ref[jj]
        off = s * BS
        kc[pl.ds(jj * BS, BS), :] = k_ref[pl.ds(off, BS), :]
        vc[pl.ds(jj * BS, BS), :] = v_ref[pl.ds(off, BS), :]
        kpos[0:1, jj * BS:(jj + 1) * BS] = (
            jax.lax.broadcasted_iota(jnp.int32, (1, BS), 1).astype(jnp.float32)
            + s.astype(jnp.float32) * BS)
    scale = 1.0 / math.sqrt(DH)
    kposc = kpos[0:1, :]
    for ti in range(T // _QT):
        base = ti * _QT
        qt = q_ref[base:base + _QT, :]
        kself = k_ref[base:base + _QT, :]
        vself = v_ref[base:base + _QT, :]
        qpos_i = (base
                  + jax.lax.broadcasted_iota(jnp.int32, (_QT, 1), 0))
        qpos = qpos_i.astype(jnp.float32)
        qbs = ((qpos_i // BS) * BS).astype(jnp.float32)
        kpos2 = (base
                 + jax.lax.broadcasted_iota(jnp.int32, (1, _QT), 1)
                 ).astype(jnp.float32)
        m1 = kposc < qbs
        m2 = (kpos2 <= qpos) & (kpos2 >= qbs)
        for hh in range(2):
            c = hh * DH
            qh = qt[:, c:c + DH]
            s1 = jax.lax.dot_general(
                qh, kc[:, c:c + DH], (((1,), (1,)), ((), ())),
                preferred_element_type=jnp.float32) * scale
            s2 = jax.lax.dot_general(
                qh, kself[:, c:c + DH], (((1,), (1,)), ((), ())),
                preferred_element_type=jnp.float32) * scale
            s1 = jnp.where(m1, s1, jnp.float32(-1e9))
            s2 = jnp.where(m2, s2, jnp.float32(-1e9))
            mx = jnp.maximum(jnp.max(s1, axis=1, keepdims=True),
                             jnp.max(s2, axis=1, keepdims=True))
            p1 = jnp.exp(s1 - mx)
            p2 = jnp.exp(s2 - mx)
            den = (jnp.sum(p1, axis=1, keepdims=True)
                   + jnp.sum(p2, axis=1, keepdims=True))
            oh = (jnp.dot(p1, vc[:, c:c + DH],
                          preferred_element_type=jnp.float32)
                  + jnp.dot(p2, vself[:, c:c + DH],
                            preferred_element_type=jnp.float32)) / den
            o_ref[base:base + _QT, c:c + DH] = oh


def _attn(sel_i, q, k, v):
    spec = pl.BlockSpec((T, 128), lambda j, sel: (0, j))
    grid_spec = pltpu.PrefetchScalarGridSpec(
        num_scalar_prefetch=1,
        grid=(H // 2,),
        in_specs=[spec, spec, spec],
        out_specs=pl.BlockSpec((T, 128), lambda j, sel: (0, j)),
        scratch_shapes=[
            pltpu.VMEM((_NSEL, 128), jnp.float32),
            pltpu.VMEM((_NSEL, 128), jnp.float32),
            pltpu.VMEM((1, _NSEL), jnp.float32),
        ],
    )
    return pl.pallas_call(
        _attn_kernel,
        grid_spec=grid_spec,
        out_shape=jax.ShapeDtypeStruct((T, D), jnp.float32),
    )(sel_i, q, k, v)


def _mm_res_kernel(a_ref, w_ref, r_ref, o_ref):
    o_ref[...] = (jnp.dot(a_ref[...], w_ref[...],
                          preferred_element_type=jnp.float32) + r_ref[...])


def _mm_res(a, w, res, bn):
    kdim = a.shape[1]
    n = w.shape[1]
    return pl.pallas_call(
        _mm_res_kernel,
        grid=(n // bn,),
        out_shape=jax.ShapeDtypeStruct((T, n), jnp.float32),
        in_specs=[
            pl.BlockSpec((T, kdim), lambda j: (0, 0)),
            pl.BlockSpec((kdim, bn), lambda j: (0, j)),
            pl.BlockSpec((T, bn), lambda j: (0, j)),
        ],
        out_specs=pl.BlockSpec((T, bn), lambda j: (0, j)),
    )(a, w, res)


def _ff1_kernel(h_ref, w1_ref, w3_ref, o_ref):
    h = h_ref[...]
    g = jnp.dot(h, w1_ref[...], preferred_element_type=jnp.float32)
    u = jnp.dot(h, w3_ref[...], preferred_element_type=jnp.float32)
    o_ref[...] = g * (1.0 / (1.0 + jnp.exp(-g))) * u


def _ff1(h2, w1l, w3l):
    bn = 512
    wspec = pl.BlockSpec((D, bn), lambda j: (0, j))
    return pl.pallas_call(
        _ff1_kernel,
        grid=(FF // bn,),
        out_shape=jax.ShapeDtypeStruct((T, FF), jnp.float32),
        in_specs=[pl.BlockSpec((T, D), lambda j: (0, 0)), wspec, wspec],
        out_specs=pl.BlockSpec((T, bn), lambda j: (0, j)),
    )(h2, w1l, w3l)


def _head_kernel(h_ref, e_ref, o_ref):
    o_ref[...] = jax.lax.dot_general(
        h_ref[...], e_ref[...], (((1,), (1,)), ((), ())),
        preferred_element_type=jnp.float32)


def _head(hf, embed):
    bn = 640
    vocab = embed.shape[0]
    return pl.pallas_call(
        _head_kernel,
        grid=(vocab // bn,),
        out_shape=jax.ShapeDtypeStruct((T, vocab), jnp.float32),
        in_specs=[
            pl.BlockSpec((T, D), lambda j: (0, 0)),
            pl.BlockSpec((bn, D), lambda j: (j, 0)),
        ],
        out_specs=pl.BlockSpec((T, bn), lambda j: (0, j)),
    )(hf, embed)


def kernel(input_ids, embed, wq, wk, wv, wo, w1, w2, w3, n1, n2, nf):
    x = _gather_sc(embed, input_ids.astype(jnp.int32))
    sel = _select(x)
    sel_i = sel.reshape(TOPK_BLOCKS).astype(jnp.int32)
    for l in range(wq.shape[0]):
        h = _rmsnorm(x, n1[l])
        q, k, v = _qkv(h, wq[l], wk[l], wv[l])
        o = _attn(sel_i, q, k, v)
        x = _mm_res(o, wo[l], x, bn=512)
        h2 = _rmsnorm(x, n2[l])
        u = _ff1(h2, w1[l], w3[l])
        x = _mm_res(u, w2[l], x, bn=256)
    hf = _rmsnorm(x, nf)
    logits = _head(hf, embed)
    return logits[None]


# speedup vs baseline: 1.4065x; 1.0156x over previous
import functools
import math

import numpy as np
import jax
import jax.numpy as jnp
from jax.experimental import pallas as pl
from jax.experimental.pallas import tpu as pltpu
from jax.experimental.pallas import tpu_sc as plsc

D = 1024
H = 16
DH = 64
FF = 4096
T = 2048
NUM_BLOCKS = 32
TOPK_BLOCKS = 8
BS = T // NUM_BLOCKS
LOW_PASS = 0.25


def _build_consts():
    fn = T // 2 + 1
    cutoff = max(1, int(LOW_PASS * fn))
    t = np.arange(T, dtype=np.float64)
    cols = [np.full(T, 1.0 / np.sqrt(T))]
    for f in range(1, cutoff):
        w = 2.0 * np.pi * f * t / T
        cols.append(np.sqrt(2.0 / T) * np.cos(w))
        cols.append(np.sqrt(2.0 / T) * np.sin(w))
    A = np.stack(cols, axis=1)
    pad = (-A.shape[1]) % 128
    A = np.concatenate([A, np.zeros((T, pad))], axis=1)
    SB = np.zeros((T, NUM_BLOCKS))
    SB[np.arange(T), np.arange(T) // BS] = 1.0 / BS
    inv_freq = 1.0 / (10000.0 ** (np.arange(0, DH, 2, dtype=np.float64) / DH))
    freqs = t[:, None] * inv_freq[None, :]
    emb = np.concatenate([freqs, freqs], axis=1)
    cos2 = np.tile(np.cos(emb), (1, 2))
    sin2 = np.tile(np.sin(emb), (1, 2))
    f32 = np.float32
    return A.astype(f32), A.T.copy().astype(f32), SB.astype(f32), cos2.astype(f32), sin2.astype(f32)


_A_NP, _AT_NP, _SB_NP, _COS2_NP, _SIN2_NP = _build_consts()


def _gather_sc(embed, ids):
    win = 128
    split = 4
    dsub = D // split
    ids4 = (ids.reshape(T, 1) * split
            + jnp.arange(split, dtype=jnp.int32).reshape(1, split)
            ).reshape(1, T * split)
    table = embed.reshape(embed.shape[0] * split, dsub)
    mesh = plsc.VectorSubcoreMesh(core_axis_name="c", subcore_axis_name="s")

    @functools.partial(
        pl.kernel,
        out_type=jax.ShapeDtypeStruct((T * split, dsub), embed.dtype),
        mesh=mesh,
    )
    def body(x_hbm, i_hbm, o_hbm):
        def inner(i_vmem, o_vmem):
            pltpu.sync_copy(x_hbm.at[i_vmem.at[0]], o_vmem)

        pltpu.emit_pipeline(
            inner,
            grid=(T * split // win,),
            in_specs=[pl.BlockSpec((1, win), lambda i: (0, i))],
            out_specs=[pl.BlockSpec((win, dsub), lambda i: (i, 0))],
            core_axis_name=("c", "s"),
            dimension_semantics=(pltpu.PARALLEL,),
        )(i_hbm, o_hbm)

    return body(table, ids4).reshape(T, D)


def _select_kernel(x_ref, a_ref, at_ref, sb_ref, sel_ref):
    x = x_ref[...]
    g = jnp.dot(at_ref[...], x, preferred_element_type=jnp.float32)
    lp = jnp.dot(a_ref[...], g, preferred_element_type=jnp.float32)
    nlp = jnp.sqrt(jnp.sum(lp * lp, axis=1, keepdims=True))
    nx = jnp.sqrt(jnp.sum(x * x, axis=1, keepdims=True))
    rho = nlp / (nx + 1e-6)
    bq = jax.lax.dot_general(
        rho, sb_ref[...], (((0,), (0,)), ((), ())),
        preferred_element_type=jnp.float32)
    iota = jax.lax.broadcasted_iota(
        jnp.int32, (1, NUM_BLOCKS), 1).astype(jnp.float32)
    iota8 = jax.lax.broadcasted_iota(
        jnp.int32, (1, TOPK_BLOCKS), 1).astype(jnp.float32)
    sel = jnp.zeros((1, TOPK_BLOCKS), jnp.float32)
    b = bq
    for j in range(TOPK_BLOCKS):
        mx = jnp.max(b, axis=1, keepdims=True)
        idx = jnp.min(jnp.where(b == mx, iota, jnp.float32(1e9)), axis=1,
                      keepdims=True)
        b = jnp.where(iota == idx, jnp.float32(-1.0), b)
        sel = sel + idx * (iota8 == jnp.float32(j)).astype(jnp.float32)
    sel_ref[...] = sel


def _select(x):
    consts = (jnp.asarray(_A_NP), jnp.asarray(_AT_NP), jnp.asarray(_SB_NP))
    return pl.pallas_call(
        _select_kernel,
        out_shape=jax.ShapeDtypeStruct((1, TOPK_BLOCKS), jnp.float32),
        in_specs=[
            pl.BlockSpec((T, D), lambda: (0, 0)),
            pl.BlockSpec((T, 512), lambda: (0, 0)),
            pl.BlockSpec((512, T), lambda: (0, 0)),
            pl.BlockSpec((T, NUM_BLOCKS), lambda: (0, 0)),
        ],
        out_specs=pl.BlockSpec((1, TOPK_BLOCKS), lambda: (0, 0)),
    )(x, *consts)


def _rmsnorm_kernel(x_ref, w_ref, o_ref):
    x = x_ref[...]
    inv = jax.lax.rsqrt(jnp.mean(x * x, axis=1, keepdims=True) + 1e-6)
    o_ref[...] = (x * inv * w_ref[...]).astype(o_ref.dtype)


def _rmsnorm(x, w, out_dtype=jnp.float32):
    return pl.pallas_call(
        _rmsnorm_kernel,
        grid=(8,),
        out_shape=jax.ShapeDtypeStruct((T, D), out_dtype),
        in_specs=[
            pl.BlockSpec((T // 8, D), lambda i: (i, 0)),
            pl.BlockSpec((1, D), lambda i: (0, 0)),
        ],
        out_specs=pl.BlockSpec((T // 8, D), lambda i: (i, 0)),
    )(x, w.reshape(1, D))


def _rot_half2(s):
    return jnp.concatenate(
        [-s[:, 32:64], s[:, 0:32], -s[:, 96:128], s[:, 64:96]], axis=1)


def _qkv_kernel(h_ref, wq_ref, wk_ref, wv_ref, cos_ref, sin_ref,
                q_ref, k_ref, v_ref):
    h = h_ref[...]
    cos = cos_ref[...]
    sin = sin_ref[...]
    q = jnp.dot(h, wq_ref[...], preferred_element_type=jnp.float32)
    k = jnp.dot(h, wk_ref[...], preferred_element_type=jnp.float32)
    v = jnp.dot(h, wv_ref[...], preferred_element_type=jnp.float32)
    q_ref[...] = q * cos + _rot_half2(q) * sin
    k_ref[...] = k * cos + _rot_half2(k) * sin
    v_ref[...] = v


def _qkv(h, wq, wk, wv):
    cos2, sin2 = jnp.asarray(_COS2_NP), jnp.asarray(_SIN2_NP)
    sds = jax.ShapeDtypeStruct((T, D), jnp.float32)
    wspec = pl.BlockSpec((D, 128), lambda j: (0, j))
    ospec = pl.BlockSpec((T, 128), lambda j: (0, j))
    return pl.pallas_call(
        _qkv_kernel,
        grid=(D // 128,),
        out_shape=(sds, sds, sds),
        in_specs=[
            pl.BlockSpec((T, D), lambda j: (0, 0)),
            wspec, wspec, wspec,
            pl.BlockSpec((T, 128), lambda j: (0, 0)),
            pl.BlockSpec((T, 128), lambda j: (0, 0)),
        ],
        out_specs=(ospec, ospec, ospec),
    )(h, wq, wk, wv, cos2, sin2)


_NSEL = TOPK_BLOCKS * BS
_QT = 256


def _attn_kernel(sel_ref, q_ref, k_ref, v_ref, o_ref, kc, vc, kpos):
    for jj in range(TOPK_BLOCKS):
        s = sel_ref[jj]
        off = s * BS
        kc[pl.ds(jj * BS, BS), :] = k_ref[pl.ds(off, BS), :]
        vc[pl.ds(jj * BS, BS), :] = v_ref[pl.ds(off, BS), :]
        kpos[0:1, jj * BS:(jj + 1) * BS] = (
            jax.lax.broadcasted_iota(jnp.int32, (1, BS), 1).astype(jnp.float32)
            + s.astype(jnp.float32) * BS)
    scale = 1.0 / math.sqrt(DH)
    kposc = kpos[0:1, :]
    for ti in range(T // _QT):
        base = ti * _QT
        qt = q_ref[base:base + _QT, :]
        kself = k_ref[base:base + _QT, :]
        vself = v_ref[base:base + _QT, :]
        qpos_i = (base
                  + jax.lax.broadcasted_iota(jnp.int32, (_QT, 1), 0))
        qpos = qpos_i.astype(jnp.float32)
        qbs = ((qpos_i // BS) * BS).astype(jnp.float32)
        kpos2 = (base
                 + jax.lax.broadcasted_iota(jnp.int32, (1, _QT), 1)
                 ).astype(jnp.float32)
        m1 = kposc < qbs
        m2 = (kpos2 <= qpos) & (kpos2 >= qbs)
        for hh in range(2):
            c = hh * DH
            qh = qt[:, c:c + DH]
            s1 = jax.lax.dot_general(
                qh, kc[:, c:c + DH], (((1,), (1,)), ((), ())),
                preferred_element_type=jnp.float32) * scale
            s2 = jax.lax.dot_general(
                qh, kself[:, c:c + DH], (((1,), (1,)), ((), ())),
                preferred_element_type=jnp.float32) * scale
            s1 = jnp.where(m1, s1, jnp.float32(-1e9))
            s2 = jnp.where(m2, s2, jnp.float32(-1e9))
            mx = jnp.maximum(jnp.max(s1, axis=1, keepdims=True),
                             jnp.max(s2, axis=1, keepdims=True))
            p1 = jnp.exp(s1 - mx)
            p2 = jnp.exp(s2 - mx)
            den = (jnp.sum(p1, axis=1, keepdims=True)
                   + jnp.sum(p2, axis=1, keepdims=True))
            oh = (jnp.dot(p1, vc[:, c:c + DH],
                          preferred_element_type=jnp.float32)
                  + jnp.dot(p2, vself[:, c:c + DH],
                            preferred_element_type=jnp.float32)) / den
            o_ref[base:base + _QT, c:c + DH] = oh


def _attn(sel_i, q, k, v):
    spec = pl.BlockSpec((T, 128), lambda j, sel: (0, j))
    grid_spec = pltpu.PrefetchScalarGridSpec(
        num_scalar_prefetch=1,
        grid=(H // 2,),
        in_specs=[spec, spec, spec],
        out_specs=pl.BlockSpec((T, 128), lambda j, sel: (0, j)),
        scratch_shapes=[
            pltpu.VMEM((_NSEL, 128), jnp.float32),
            pltpu.VMEM((_NSEL, 128), jnp.float32),
            pltpu.VMEM((1, _NSEL), jnp.float32),
        ],
    )
    return pl.pallas_call(
        _attn_kernel,
        grid_spec=grid_spec,
        out_shape=jax.ShapeDtypeStruct((T, D), jnp.float32),
    )(sel_i, q, k, v)


def _mm_res_kernel(a_ref, w_ref, r_ref, o_ref):
    a = a_ref[...]
    w = w_ref[...].astype(a.dtype)
    o_ref[...] = (jnp.dot(a, w, preferred_element_type=jnp.float32)
                  + r_ref[...])


def _mm_res(a, w, res, bn):
    kdim = a.shape[1]
    n = w.shape[1]
    return pl.pallas_call(
        _mm_res_kernel,
        grid=(n // bn,),
        out_shape=jax.ShapeDtypeStruct((T, n), jnp.float32),
        in_specs=[
            pl.BlockSpec((T, kdim), lambda j: (0, 0)),
            pl.BlockSpec((kdim, bn), lambda j: (0, j)),
            pl.BlockSpec((T, bn), lambda j: (0, j)),
        ],
        out_specs=pl.BlockSpec((T, bn), lambda j: (0, j)),
    )(a, w, res)


def _ff1_kernel(h_ref, w1_ref, w3_ref, o_ref):
    h = h_ref[...]
    g = jnp.dot(h, w1_ref[...].astype(h.dtype),
                preferred_element_type=jnp.float32)
    u = jnp.dot(h, w3_ref[...].astype(h.dtype),
                preferred_element_type=jnp.float32)
    o_ref[...] = (g * (1.0 / (1.0 + jnp.exp(-g))) * u).astype(o_ref.dtype)


def _ff1(h2, w1l, w3l):
    bn = 512
    wspec = pl.BlockSpec((D, bn), lambda j: (0, j))
    return pl.pallas_call(
        _ff1_kernel,
        grid=(FF // bn,),
        out_shape=jax.ShapeDtypeStruct((T, FF), jnp.bfloat16),
        in_specs=[pl.BlockSpec((T, D), lambda j: (0, 0)), wspec, wspec],
        out_specs=pl.BlockSpec((T, bn), lambda j: (0, j)),
    )(h2, w1l, w3l)


def _head_kernel(h_ref, e_ref, o_ref):
    o_ref[...] = jax.lax.dot_general(
        h_ref[...], e_ref[...].astype(h_ref.dtype), (((1,), (1,)), ((), ())),
        preferred_element_type=jnp.float32)


def _head(hf, embed):
    bn = 640
    vocab = embed.shape[0]
    return pl.pallas_call(
        _head_kernel,
        grid=(vocab // bn,),
        out_shape=jax.ShapeDtypeStruct((T, vocab), jnp.float32),
        in_specs=[
            pl.BlockSpec((T, D), lambda j: (0, 0)),
            pl.BlockSpec((bn, D), lambda j: (j, 0)),
        ],
        out_specs=pl.BlockSpec((T, bn), lambda j: (0, j)),
    )(hf, embed)


def kernel(input_ids, embed, wq, wk, wv, wo, w1, w2, w3, n1, n2, nf):
    x = _gather_sc(embed, input_ids.astype(jnp.int32))
    sel = _select(x)
    sel_i = sel.reshape(TOPK_BLOCKS).astype(jnp.int32)
    for l in range(wq.shape[0]):
        h = _rmsnorm(x, n1[l])
        q, k, v = _qkv(h, wq[l], wk[l], wv[l])
        o = _attn(sel_i, q, k, v)
        x = _mm_res(o, wo[l], x, bn=512)
        h2 = _rmsnorm(x, n2[l], out_dtype=jnp.bfloat16)
        u = _ff1(h2, w1[l], w3[l])
        x = _mm_res(u, w2[l], x, bn=256)
    hf = _rmsnorm(x, nf, out_dtype=jnp.bfloat16)
    logits = _head(hf, embed)
    return logits[None]
